# Initial kernel scaffold; baseline (speedup 1.0000x reference)
import functools

import jax
import jax.numpy as jnp
from jax import lax
from jax.experimental import pallas as pl
from jax.experimental.pallas import tpu as pltpu
from jax.experimental.pallas import tpu_sc as plsc

N = 10000
E = 320000
B = 16
D = 128
O = 128

NC = 2
NS = 16
NW = NC * NS
EW = E // NW
K = 128
CHP = 80
EWP = CHP * K
PAD = EWP - EW
RPS = N // NS

_GATHER_DNUMS = lax.GatherDimensionNumbers(
    offset_dims=(), collapsed_slice_dims=(0,), start_index_map=(0,))


def _film_body(nf_ref, cond_ref, bid_ref, wc_ref, bc_ref, wl_ref, ew_ref,
               ep_ref, h_ref, w_ref):
    cond = cond_ref[...]
    wc = wc_ref[...]
    gb = lax.dot_general(cond, wc, (((1,), (1,)), ((), ())),
                         preferred_element_type=jnp.float32)
    gb = gb + jnp.concatenate([bc_ref[0:1, :], bc_ref[1:2, :]], axis=1)
    gamma = gb[:, :O] + 1.0
    beta = gb[:, O:]

    h = lax.dot_general(nf_ref[...], wl_ref[...], (((1,), (1,)), ((), ())),
                        preferred_element_type=jnp.float32)
    mu = jnp.mean(h, axis=1, keepdims=True)
    d = h - mu
    var = jnp.mean(d * d, axis=1, keepdims=True)
    hn = d * lax.rsqrt(var + 1e-5)

    bid = bid_ref[...]
    oh = (bid == lax.broadcasted_iota(jnp.int32, (1, B), 1))
    oh = oh.astype(jnp.float32)
    g = jnp.dot(oh, gamma, preferred_element_type=jnp.float32)
    b = jnp.dot(oh, beta, preferred_element_type=jnp.float32)
    h_ref[...] = jnp.maximum(hn * g + b, 0.0)

    w_ref[...] = ew_ref[...] * ep_ref[...]


_BN = 1000
_BE = E // 128 // 10

_film_call = pl.pallas_call(
    _film_body,
    grid=(N // _BN,),
    in_specs=[
        pl.BlockSpec((_BN, D), lambda i: (i, 0)),
        pl.BlockSpec((B, D), lambda i: (0, 0)),
        pl.BlockSpec((_BN, 1), lambda i: (i, 0)),
        pl.BlockSpec((2 * O, D), lambda i: (0, 0)),
        pl.BlockSpec((2, O), lambda i: (0, 0)),
        pl.BlockSpec((O, D), lambda i: (0, 0)),
        pl.BlockSpec((_BE, 128), lambda i: (i, 0)),
        pl.BlockSpec((_BE, 128), lambda i: (i, 0)),
    ],
    out_specs=[
        pl.BlockSpec((_BN, O), lambda i: (i, 0)),
        pl.BlockSpec((_BE, 128), lambda i: (i, 0)),
    ],
    out_shape=[
        jax.ShapeDtypeStruct((N, O), jnp.float32),
        jax.ShapeDtypeStruct((E // 128, 128), jnp.float32),
    ],
)


def _mp_body(h_hbm, jid_hbm, iid_hbm, w_hbm, out_hbm,
             jid_v, iid_v, w_v, rows_v, accum):
    c = lax.axis_index("c")
    s = lax.axis_index("s")

    zero16 = jnp.zeros((16,), jnp.float32)

    @pl.loop(0, 125)
    def _zero_rows(r):
        for f in range(8):
            rows_v[r, pl.ds(f * 16, 16)] = zero16

    base = s * RPS
    for kk in range(5):
        pltpu.sync_copy(rows_v.at[pl.ds(0, 125)],
                        accum.at[pl.ds(base + kk * 125, 125)])
    plsc.subcore_barrier()

    pltpu.sync_copy(jid_hbm.at[c, s], jid_v)
    pltpu.sync_copy(iid_hbm.at[c, s], iid_v)
    pltpu.sync_copy(w_hbm.at[c, s], w_v)

    @pl.loop(0, CHP)
    def _chunk(ch):
        pltpu.sync_copy(h_hbm.at[jid_v.at[ch]], rows_v)

        @pl.loop(0, 8)
        def _group(g):
            w16 = w_v[ch, pl.ds(g * 16, 16)]
            for le in range(16):
                splat = lax.gather(
                    w16, jnp.full((16, 1), le, jnp.int32), _GATHER_DNUMS,
                    (1,), mode=lax.GatherScatterMode.PROMISE_IN_BOUNDS)
                e = g * 16 + le
                for f in range(8):
                    rows_v[e, pl.ds(f * 16, 16)] = (
                        rows_v[e, pl.ds(f * 16, 16)] * splat)

        pltpu.sync_copy(rows_v, accum.at[iid_v.at[ch]], add=True)

    plsc.subcore_barrier()

    pltpu.sync_copy(accum.at[pl.ds(base, RPS)],
                    out_hbm.at[c, pl.ds(base, RPS)])


_mp_call = pl.kernel(
    _mp_body,
    out_type=jax.ShapeDtypeStruct((NC, N, O), jnp.float32),
    mesh=plsc.VectorSubcoreMesh(core_axis_name="c", subcore_axis_name="s"),
    scratch_types=[
        pltpu.VMEM((CHP, K), jnp.int32),
        pltpu.VMEM((CHP, K), jnp.int32),
        pltpu.VMEM((CHP, K), jnp.float32),
        pltpu.VMEM((K, O), jnp.float32),
        pltpu.VMEM_SHARED((N, O), jnp.float32),
    ],
)


def _fin_body(p_ref, o_ref):
    o_ref[...] = jnp.maximum(p_ref[0] + p_ref[1], 0.0)


_fin_call = pl.pallas_call(
    _fin_body,
    grid=(N // _BN,),
    in_specs=[pl.BlockSpec((NC, _BN, O), lambda i: (0, i, 0))],
    out_specs=pl.BlockSpec((_BN, O), lambda i: (i, 0)),
    out_shape=jax.ShapeDtypeStruct((N, O), jnp.float32),
)


def kernel(node_feats, cond_feats, batch_ids, edge_weights, edge_params,
           node_j_ids, node_i_ids, W_cond, b_cond, W_lin):
    bid2 = batch_ids.reshape(N, 1)
    bc2 = b_cond.reshape(2, O)
    ew2 = edge_weights.reshape(E // 128, 128)
    ep2 = edge_params.reshape(E // 128, 128)

    h, w = _film_call(node_feats, cond_feats, bid2, W_cond, bc2, W_lin,
                      ew2, ep2)

    spread = (jnp.arange(NW * PAD, dtype=jnp.int32) % N).reshape(NW, PAD)
    wp = jnp.concatenate(
        [w.reshape(NW, EW), jnp.zeros((NW, PAD), jnp.float32)],
        axis=1).reshape(NC, NS, CHP, K)
    jp = jnp.concatenate([node_j_ids.reshape(NW, EW), spread],
                         axis=1).reshape(NC, NS, CHP, K)
    ip = jnp.concatenate([node_i_ids.reshape(NW, EW), spread],
                         axis=1).reshape(NC, NS, CHP, K)

    partials = _mp_call(h, jp, ip, wp)
    return _fin_call(partials)


# speedup vs baseline: 6.3875x; 6.3875x over previous
import functools

import jax
import jax.numpy as jnp
from jax import lax
from jax.experimental import pallas as pl
from jax.experimental.pallas import tpu as pltpu
from jax.experimental.pallas import tpu_sc as plsc

N = 10000
E = 320000
B = 16
D = 128
O = 128

NC = 2
NS = 16
NW = NC * NS
EW = E // NW
K = 128
CHP = 80
EWP = CHP * K
PAD = EWP - EW
NP = 10240
RPS = NP // NS

_GATHER_DNUMS = lax.GatherDimensionNumbers(
    offset_dims=(), collapsed_slice_dims=(0,), start_index_map=(0,))


def _film_body(nf_ref, cond_ref, bid_ref, wc_ref, bc_ref, wl_ref, ew_ref,
               ep_ref, h_ref, w_ref):
    cond = cond_ref[...]
    wc = wc_ref[...]
    gb = lax.dot_general(cond, wc, (((1,), (1,)), ((), ())),
                         preferred_element_type=jnp.float32)
    gb = gb + jnp.concatenate([bc_ref[0:1, :], bc_ref[1:2, :]], axis=1)
    gamma = gb[:, :O] + 1.0
    beta = gb[:, O:]

    h = lax.dot_general(nf_ref[...], wl_ref[...], (((1,), (1,)), ((), ())),
                        preferred_element_type=jnp.float32)
    mu = jnp.mean(h, axis=1, keepdims=True)
    d = h - mu
    var = jnp.mean(d * d, axis=1, keepdims=True)
    hn = d * lax.rsqrt(var + 1e-5)

    bid = bid_ref[...]
    oh = (bid == lax.broadcasted_iota(jnp.int32, (1, B), 1))
    oh = oh.astype(jnp.float32)
    g = jnp.dot(oh, gamma, preferred_element_type=jnp.float32)
    b = jnp.dot(oh, beta, preferred_element_type=jnp.float32)
    h_ref[...] = jnp.maximum(hn * g + b, 0.0)

    w_ref[...] = ew_ref[...] * ep_ref[...]


_BN = 1000
_ER = 320
_BE = _ER // 10

_film_call = pl.pallas_call(
    _film_body,
    grid=(N // _BN,),
    in_specs=[
        pl.BlockSpec((_BN, D), lambda i: (i, 0)),
        pl.BlockSpec((B, D), lambda i: (0, 0)),
        pl.BlockSpec((_BN, 1), lambda i: (i, 0)),
        pl.BlockSpec((2 * O, D), lambda i: (0, 0)),
        pl.BlockSpec((2, O), lambda i: (0, 0)),
        pl.BlockSpec((O, D), lambda i: (0, 0)),
        pl.BlockSpec((_BE, 1000), lambda i: (i, 0)),
        pl.BlockSpec((_BE, 1000), lambda i: (i, 0)),
    ],
    out_specs=[
        pl.BlockSpec((_BN, O), lambda i: (i, 0)),
        pl.BlockSpec((_BE, 1000), lambda i: (i, 0)),
    ],
    out_shape=[
        jax.ShapeDtypeStruct((N, O), jnp.float32),
        jax.ShapeDtypeStruct((_ER, 1000), jnp.float32),
    ],
)


def _mp_body(h_hbm, jid_hbm, iid_hbm, w_hbm, out_hbm,
             jid_v, iid_v, w_v, rows_v, accum):
    c = lax.axis_index("c")
    s = lax.axis_index("s")

    zero16 = jnp.zeros((16,), jnp.float32)

    @pl.loop(0, K)
    def _zero_rows(r):
        for f in range(8):
            rows_v[r, pl.ds(f * 16, 16)] = zero16

    base = s * RPS
    for kk in range(RPS // K):
        pltpu.sync_copy(rows_v.at[pl.ds(0, K)],
                        accum.at[pl.ds(base + kk * K, K)])
    plsc.subcore_barrier()

    pltpu.sync_copy(jid_hbm.at[c, s], jid_v)
    pltpu.sync_copy(iid_hbm.at[c, s], iid_v)
    pltpu.sync_copy(w_hbm.at[c, s], w_v)

    @pl.loop(0, CHP)
    def _chunk(ch):
        pltpu.sync_copy(h_hbm.at[jid_v.at[ch]], rows_v)

        @pl.loop(0, 8)
        def _group(g):
            w16 = w_v[ch, pl.ds(g * 16, 16)]
            for le in range(16):
                splat = lax.gather(
                    w16, jnp.full((16, 1), le, jnp.int32), _GATHER_DNUMS,
                    (1,), mode=lax.GatherScatterMode.PROMISE_IN_BOUNDS)
                e = g * 16 + le
                for f in range(8):
                    rows_v[e, pl.ds(f * 16, 16)] = (
                        rows_v[e, pl.ds(f * 16, 16)] * splat)

        pltpu.sync_copy(rows_v, accum.at[iid_v.at[ch]], add=True)

    plsc.subcore_barrier()

    pltpu.sync_copy(accum.at[pl.ds(base, RPS)],
                    out_hbm.at[c, pl.ds(base, RPS)])


_mp_call = pl.kernel(
    _mp_body,
    out_type=jax.ShapeDtypeStruct((NC, NP, O), jnp.float32),
    mesh=plsc.VectorSubcoreMesh(core_axis_name="c", subcore_axis_name="s"),
    scratch_types=[
        pltpu.VMEM((CHP, K), jnp.int32),
        pltpu.VMEM((CHP, K), jnp.int32),
        pltpu.VMEM((CHP, K), jnp.float32),
        pltpu.VMEM((K, O), jnp.float32),
        pltpu.VMEM_SHARED((NP, O), jnp.float32),
    ],
)


def _fin_body(p_ref, o_ref):
    p = p_ref[...]
    o_ref[...] = jnp.maximum(p[0, :N, :] + p[1, :N, :], 0.0)


_fin_call = pl.pallas_call(
    _fin_body,
    in_specs=[pl.BlockSpec((NC, NP, O), lambda: (0, 0, 0))],
    out_specs=pl.BlockSpec((N, O), lambda: (0, 0)),
    out_shape=jax.ShapeDtypeStruct((N, O), jnp.float32),
)


def kernel(node_feats, cond_feats, batch_ids, edge_weights, edge_params,
           node_j_ids, node_i_ids, W_cond, b_cond, W_lin):
    bid2 = batch_ids.reshape(N, 1)
    bc2 = b_cond.reshape(2, O)
    ew2 = edge_weights.reshape(_ER, 1000)
    ep2 = edge_params.reshape(_ER, 1000)

    h, w = _film_call(node_feats, cond_feats, bid2, W_cond, bc2, W_lin,
                      ew2, ep2)

    spread = (jnp.arange(NW * PAD, dtype=jnp.int32) % N).reshape(NW, PAD)
    wp = jnp.concatenate(
        [w.reshape(NW, EW), jnp.zeros((NW, PAD), jnp.float32)],
        axis=1).reshape(NC, NS, CHP, K)
    jp = jnp.concatenate([node_j_ids.reshape(NW, EW), spread],
                         axis=1).reshape(NC, NS, CHP, K)
    ip = jnp.concatenate([node_i_ids.reshape(NW, EW), spread],
                         axis=1).reshape(NC, NS, CHP, K)

    partials = _mp_call(h, jp, ip, wp)
    return _fin_call(partials)


# speedup vs baseline: 8.7251x; 1.3660x over previous
import functools

import jax
import jax.numpy as jnp
from jax import lax
from jax.experimental import pallas as pl
from jax.experimental.pallas import tpu as pltpu
from jax.experimental.pallas import tpu_sc as plsc

N = 10000
E = 320000
B = 16
D = 128
O = 128

NC = 2
NS = 16
NW = NC * NS
EW = E // NW
K = 128
CHP = 80
GRP = 8
NG = CHP // GRP
EWP = CHP * K
PAD = EWP - EW
NP = 10240
RPS = NP // NS

_GATHER_DNUMS = lax.GatherDimensionNumbers(
    offset_dims=(), collapsed_slice_dims=(0,), start_index_map=(0,))


def _film_body(nf_ref, cond_ref, bid_ref, wc_ref, bc_ref, wl_ref, ew_ref,
               ep_ref, h_ref, w_ref):
    cond = cond_ref[...]
    wc = wc_ref[...]
    gb = lax.dot_general(cond, wc, (((1,), (1,)), ((), ())),
                         preferred_element_type=jnp.float32)
    gb = gb + jnp.concatenate([bc_ref[0:1, :], bc_ref[1:2, :]], axis=1)
    gamma = gb[:, :O] + 1.0
    beta = gb[:, O:]

    h = lax.dot_general(nf_ref[...], wl_ref[...], (((1,), (1,)), ((), ())),
                        preferred_element_type=jnp.float32)
    mu = jnp.mean(h, axis=1, keepdims=True)
    d = h - mu
    var = jnp.mean(d * d, axis=1, keepdims=True)
    hn = d * lax.rsqrt(var + 1e-5)

    bid = bid_ref[...]
    oh = (bid == lax.broadcasted_iota(jnp.int32, (1, B), 1))
    oh = oh.astype(jnp.float32)
    g = jnp.dot(oh, gamma, preferred_element_type=jnp.float32)
    b = jnp.dot(oh, beta, preferred_element_type=jnp.float32)
    h_ref[...] = jnp.maximum(hn * g + b, 0.0)

    w_ref[...] = ew_ref[...] * ep_ref[...]


_BN = 1000
_ER = 320
_BE = _ER // 10

_film_call = pl.pallas_call(
    _film_body,
    grid=(N // _BN,),
    in_specs=[
        pl.BlockSpec((_BN, D), lambda i: (i, 0)),
        pl.BlockSpec((B, D), lambda i: (0, 0)),
        pl.BlockSpec((_BN, 1), lambda i: (i, 0)),
        pl.BlockSpec((2 * O, D), lambda i: (0, 0)),
        pl.BlockSpec((2, O), lambda i: (0, 0)),
        pl.BlockSpec((O, D), lambda i: (0, 0)),
        pl.BlockSpec((_BE, 1000), lambda i: (i, 0)),
        pl.BlockSpec((_BE, 1000), lambda i: (i, 0)),
    ],
    out_specs=[
        pl.BlockSpec((_BN, O), lambda i: (i, 0)),
        pl.BlockSpec((_BE, 1000), lambda i: (i, 0)),
    ],
    out_shape=[
        jax.ShapeDtypeStruct((N, O), jnp.float32),
        jax.ShapeDtypeStruct((_ER, 1000), jnp.float32),
    ],
)


def _mp_body(h_hbm, jid_hbm, iid_hbm, w_hbm, out_hbm,
             jid_v, iid_v, w_v, rows_a, rows_b, sem_a, sem_b, sem_m, accum):
    c = lax.axis_index("c")
    s = lax.axis_index("s")

    zero16 = jnp.zeros((16,), jnp.float32)

    @pl.loop(0, K)
    def _zero_rows(r):
        for f in range(8):
            rows_a[r, pl.ds(f * 16, 16)] = zero16

    base = s * RPS
    for kk in range(RPS // K):
        pltpu.sync_copy(rows_a.at[pl.ds(0, K)],
                        accum.at[pl.ds(base + kk * K, K)])
    plsc.subcore_barrier()

    def scale(rows_v, slot, kk):
        @pl.loop(0, K // 16)
        def _group(g):
            w16 = w_v[slot, kk, pl.ds(g * 16, 16)]
            for le in range(16):
                splat = lax.gather(
                    w16, jnp.full((16, 1), le, jnp.int32), _GATHER_DNUMS,
                    (1,), mode=lax.GatherScatterMode.PROMISE_IN_BOUNDS)
                e = g * 16 + le
                for f in range(8):
                    rows_v[e, pl.ds(f * 16, 16)] = (
                        rows_v[e, pl.ds(f * 16, 16)] * splat)

    def fire_meta(g, slot):
        pltpu.async_copy(jid_hbm.at[c, s, pl.ds(g * GRP, GRP)],
                         jid_v.at[slot], sem_m)
        pltpu.async_copy(iid_hbm.at[c, s, pl.ds(g * GRP, GRP)],
                         iid_v.at[slot], sem_m)
        pltpu.async_copy(w_hbm.at[c, s, pl.ds(g * GRP, GRP)],
                         w_v.at[slot], sem_m)

    def wait_meta(slot):
        pltpu.make_async_copy(jid_hbm.at[c, s, pl.ds(0, GRP)],
                              jid_v.at[slot], sem_m).wait()
        pltpu.make_async_copy(iid_hbm.at[c, s, pl.ds(0, GRP)],
                              iid_v.at[slot], sem_m).wait()
        pltpu.make_async_copy(w_hbm.at[c, s, pl.ds(0, GRP)],
                              w_v.at[slot], sem_m).wait()

    fire_meta(0, 0)
    for g in range(NG):
        slot = g % 2
        wait_meta(slot)
        if g + 1 < NG:
            fire_meta(g + 1, 1 - slot)

        pltpu.async_copy(h_hbm.at[jid_v.at[slot, 0]], rows_a, sem_a)

        @pl.loop(0, GRP // 2)
        def _pair(t):
            kk = t * 2
            pltpu.make_async_copy(h_hbm.at[jid_v.at[slot, kk]], rows_a,
                                  sem_a).wait()
            pltpu.async_copy(h_hbm.at[jid_v.at[slot, kk + 1]], rows_b, sem_b)
            scale(rows_a, slot, kk)
            pltpu.sync_copy(rows_a, accum.at[iid_v.at[slot, kk]], add=True)

            pltpu.make_async_copy(h_hbm.at[jid_v.at[slot, kk + 1]], rows_b,
                                  sem_b).wait()

            @pl.when(kk + 2 < GRP)
            def _prefetch():
                pltpu.async_copy(h_hbm.at[jid_v.at[slot, kk + 2]], rows_a,
                                 sem_a)

            scale(rows_b, slot, kk + 1)
            pltpu.sync_copy(rows_b, accum.at[iid_v.at[slot, kk + 1]],
                            add=True)

    plsc.subcore_barrier()

    pltpu.sync_copy(accum.at[pl.ds(base, RPS)],
                    out_hbm.at[c, pl.ds(base, RPS)])


_mp_call = pl.kernel(
    _mp_body,
    out_type=jax.ShapeDtypeStruct((NC, NP, O), jnp.float32),
    mesh=plsc.VectorSubcoreMesh(core_axis_name="c", subcore_axis_name="s"),
    scratch_types=[
        pltpu.VMEM((2, GRP, K), jnp.int32),
        pltpu.VMEM((2, GRP, K), jnp.int32),
        pltpu.VMEM((2, GRP, K), jnp.float32),
        pltpu.VMEM((K, O), jnp.float32),
        pltpu.VMEM((K, O), jnp.float32),
        pltpu.SemaphoreType.DMA,
        pltpu.SemaphoreType.DMA,
        pltpu.SemaphoreType.DMA,
        pltpu.VMEM_SHARED((NP, O), jnp.float32),
    ],
)


def _fin_body(p_ref, o_ref):
    p = p_ref[...]
    o_ref[...] = jnp.maximum(p[0, :N, :] + p[1, :N, :], 0.0)


_fin_call = pl.pallas_call(
    _fin_body,
    in_specs=[pl.BlockSpec((NC, NP, O), lambda: (0, 0, 0))],
    out_specs=pl.BlockSpec((N, O), lambda: (0, 0)),
    out_shape=jax.ShapeDtypeStruct((N, O), jnp.float32),
)


def kernel(node_feats, cond_feats, batch_ids, edge_weights, edge_params,
           node_j_ids, node_i_ids, W_cond, b_cond, W_lin):
    bid2 = batch_ids.reshape(N, 1)
    bc2 = b_cond.reshape(2, O)
    ew2 = edge_weights.reshape(_ER, 1000)
    ep2 = edge_params.reshape(_ER, 1000)

    h, w = _film_call(node_feats, cond_feats, bid2, W_cond, bc2, W_lin,
                      ew2, ep2)

    spread = (jnp.arange(NW * PAD, dtype=jnp.int32) % N).reshape(NW, PAD)
    wp = jnp.concatenate(
        [w.reshape(NW, EW), jnp.zeros((NW, PAD), jnp.float32)],
        axis=1).reshape(NC, NS, CHP, K)
    jp = jnp.concatenate([node_j_ids.reshape(NW, EW), spread],
                         axis=1).reshape(NC, NS, CHP, K)
    ip = jnp.concatenate([node_i_ids.reshape(NW, EW), spread],
                         axis=1).reshape(NC, NS, CHP, K)

    partials = _mp_call(h, jp, ip, wp)
    return _fin_call(partials)
